# baseline (speedup 1.0000x reference)
import functools

import jax
import jax.numpy as jnp
from jax import lax
from jax.experimental import pallas as pl
from jax.experimental.pallas import tpu as pltpu
from jax.experimental.pallas import tpu_sc as plsc

_EPS = 1e-5

_NC = 2
_NS = 16
_NW = _NC * _NS
_SUB = 128


def _sc_gather(user_ids, item_ids, user_table, item_table):
    B = user_ids.shape[0]
    D = user_table.shape[1]
    ch = B // _NW
    nchunk = ch // _SUB

    uids2 = user_ids.reshape(B // _SUB, _SUB)
    iids2 = item_ids.reshape(B // _SUB, _SUB)

    mesh = plsc.VectorSubcoreMesh(core_axis_name="c", subcore_axis_name="s")

    @functools.partial(
        pl.kernel,
        mesh=mesh,
        out_type=jax.ShapeDtypeStruct((B, 2 * D), jnp.float32),
        scratch_types=[
            pltpu.VMEM((nchunk, _SUB), jnp.int32),
            pltpu.VMEM((ch, D), jnp.float32),
            pltpu.SemaphoreType.DMA,
        ],
    )
    def gk(uids, iids, ut, it, out, idx_v, rows_v, sem):
        wid = lax.axis_index("s") * _NC + lax.axis_index("c")
        base = wid * ch
        for col, ids, tbl in ((0, uids, ut), (D, iids, it)):
            pltpu.sync_copy(ids.at[pl.ds(wid * nchunk, nchunk)], idx_v)
            handles = []
            for j in range(nchunk):
                handles.append(
                    pltpu.async_copy(
                        tbl.at[idx_v.at[j]],
                        rows_v.at[pl.ds(j * _SUB, _SUB)],
                        sem,
                    )
                )
            for h in handles:
                h.wait()
            pltpu.sync_copy(rows_v, out.at[pl.ds(base, ch), pl.ds(col, D)])

    return gk(uids2, iids2, user_table, item_table)


def _genre_body(gid_ref, gt_ref, aw_ref, ge_ref, *, G, NGP, TB):
    f32 = jnp.float32
    bf16 = jnp.bfloat16
    L = G * NGP
    gt = gt_ref[...]
    aw = aw_ref[...]
    lt_row = lax.dot_general(aw, gt, (((1,), (1,)), ((), ())),
                             preferred_element_type=f32)
    lt_tiled = jnp.concatenate([lt_row] * G, axis=1)
    rg = lax.broadcasted_iota(jnp.int32, (G, L), 0)
    rl = lax.broadcasted_iota(jnp.int32, (G, L), 1)
    R = (rg == rl // NGP).astype(bf16)
    gid_rep = lax.dot_general(gid_ref[...].astype(bf16), R,
                              (((1,), (0,)), ((), ())),
                              preferred_element_type=f32)
    lane_t = (lax.broadcasted_iota(jnp.int32, (TB, L), 1) % NGP).astype(f32)
    cmp = gid_rep == lane_t
    mlt = jnp.where(cmp, lt_tiled, 0.0)
    l5 = lax.dot_general(mlt.astype(bf16), R, (((1,), (1,)), ((), ())),
                         preferred_element_type=f32)
    m = jnp.max(l5, axis=1, keepdims=True)
    e5 = jnp.exp(l5 - m)
    w5 = e5 / jnp.sum(e5, axis=1, keepdims=True)
    w_rep = lax.dot_general(w5.astype(bf16), R, (((1,), (0,)), ((), ())),
                            preferred_element_type=f32)
    woh = jnp.where(cmp, w_rep, 0.0)
    gt5 = jnp.concatenate([gt] * G, axis=0).astype(bf16)
    ge_ref[...] = lax.dot_general(woh.astype(bf16), gt5,
                                  (((1,), (0,)), ((), ())),
                                  preferred_element_type=f32).astype(bf16)


def _mlp_body(uei_ref, gemb_ref,
              w0ui_ref, w0g_ref, b0_ref, g0_ref, be0_ref,
              w1_ref, b1_ref, g1_ref, be1_ref,
              w2_ref, b2_ref, g2_ref, be2_ref,
              wp_ref, bp_ref,
              o_ref,
              z0_sc, s0_sc, ss0_sc,
              *, B, TB, NT):
    s = pl.program_id(0)
    bf16 = jnp.bfloat16

    def mm(a, w_ref):
        return lax.dot_general(a.astype(bf16), w_ref[...].astype(bf16),
                               (((1,), (1,)), ((), ())),
                               preferred_element_type=jnp.float32)

    @pl.when(s < NT)
    def _pass0():
        row = pl.ds(pl.multiple_of(s * TB, TB), TB)
        z = mm(uei_ref[...], w0ui_ref)
        z = z + mm(gemb_ref[...], w0g_ref) + b0_ref[...]
        z0_sc[row, :] = z

        @pl.when(s == 0)
        def _():
            s0_sc[...] = jnp.zeros_like(s0_sc)
            ss0_sc[...] = jnp.zeros_like(ss0_sc)

        s0_sc[...] += jnp.sum(z, axis=0, keepdims=True)
        ss0_sc[...] += jnp.sum(z * z, axis=0, keepdims=True)

    def bn_relu(z, s_row, ss_row, g_ref, be_ref):
        mean = s_row / B
        var = ss_row / B - mean * mean
        a = lax.rsqrt(var + _EPS) * g_ref[...]
        c = be_ref[...] - mean * a
        return jnp.maximum(z * a + c, 0.0)

    @pl.when(s == NT)
    def _rest():
        h0 = bn_relu(z0_sc[...], s0_sc[...], ss0_sc[...], g0_ref, be0_ref)
        z1 = mm(h0, w1_ref) + b1_ref[...]
        h1 = bn_relu(z1, jnp.sum(z1, axis=0, keepdims=True),
                     jnp.sum(z1 * z1, axis=0, keepdims=True),
                     g1_ref, be1_ref)
        z2 = mm(h1, w2_ref) + b2_ref[...]
        h2 = bn_relu(z2, jnp.sum(z2, axis=0, keepdims=True),
                     jnp.sum(z2 * z2, axis=0, keepdims=True),
                     g2_ref, be2_ref)
        ot = lax.dot_general(wp_ref[...].astype(bf16), h2.astype(bf16),
                             (((1,), (1,)), ((), ())),
                             preferred_element_type=jnp.float32) + bp_ref[0, 0]
        o_ref[...] = (jax.nn.sigmoid(ot) * 5.0).reshape(1, 1, B)


def kernel(user_ids, item_ids, genre_ids, user_table, item_table, genre_table,
           attn_w, attn_b, W0, b0, gamma0, beta0, W1, b1, gamma1, beta1,
           W2, b2, gamma2, beta2, Wp, bp):
    B = user_ids.shape[0]
    G = genre_ids.shape[1]
    NG, DG = genre_table.shape
    D = user_table.shape[1]
    TB = 2048
    nt = B // TB
    f32 = jnp.float32

    uei = _sc_gather(user_ids.astype(jnp.int32), item_ids.astype(jnp.int32),
                     user_table, item_table)

    NGP = 24
    gt_pad = jnp.zeros((NGP, DG), f32).at[:NG].set(genre_table)
    gid = genre_ids.astype(jnp.int32)

    gemb = pl.pallas_call(
        functools.partial(_genre_body, G=G, NGP=NGP, TB=TB),
        grid=(nt,),
        in_specs=[
            pl.BlockSpec((TB, G), lambda i: (i, 0)),
            pl.BlockSpec((NGP, DG), lambda i: (0, 0)),
            pl.BlockSpec((1, DG), lambda i: (0, 0)),
        ],
        out_specs=pl.BlockSpec((TB, DG), lambda i: (i, 0)),
        out_shape=jax.ShapeDtypeStruct((B, DG), jnp.bfloat16),
    )(gid, gt_pad, attn_w)

    H0, H1, H2 = W0.shape[0], W1.shape[0], W2.shape[0]
    row = lambda v: v.reshape(1, -1)

    p0 = lambda s: (jnp.minimum(s, nt - 1), 0)
    fix = lambda s: (0, 0)
    full = lambda sh: pl.BlockSpec(sh, fix)

    out = pl.pallas_call(
        functools.partial(_mlp_body, B=B, TB=TB, NT=nt),
        grid=(nt + 1,),
        in_specs=[
            pl.BlockSpec((TB, 2 * D), p0),
            pl.BlockSpec((TB, DG), p0),
            full((H0, 2 * D)), full((H0, DG)),
            full((1, H0)), full((1, H0)), full((1, H0)),
            full((H1, H0)), full((1, H1)), full((1, H1)), full((1, H1)),
            full((H2, H1)), full((1, H2)), full((1, H2)), full((1, H2)),
            full((1, H2)), full((1, 1)),
        ],
        out_specs=pl.BlockSpec((1, 1, B), lambda s: (0, 0, 0)),
        out_shape=jax.ShapeDtypeStruct((1, 1, B), f32),
        scratch_shapes=[
            pltpu.VMEM((B, H0), f32),
            pltpu.VMEM((1, H0), f32), pltpu.VMEM((1, H0), f32),
        ],
    )(uei, gemb,
      W0[:, :2 * D], W0[:, 2 * D:],
      row(b0), row(gamma0), row(beta0),
      W1, row(b1), row(gamma1), row(beta1),
      W2, row(b2), row(gamma2), row(beta2),
      Wp, row(bp))

    return out.reshape(B)


# speedup vs baseline: 1.0656x; 1.0656x over previous
import functools

import jax
import jax.numpy as jnp
from jax import lax
from jax.experimental import pallas as pl
from jax.experimental.pallas import tpu as pltpu
from jax.experimental.pallas import tpu_sc as plsc

_EPS = 1e-5

_NC = 2
_NS = 16
_NW = _NC * _NS
_SUB = 128


def _sc_gather(user_ids, item_ids, user_table, item_table):
    B = user_ids.shape[0]
    D = user_table.shape[1]
    ch = B // _NW
    nchunk = ch // _SUB

    uids2 = user_ids.reshape(B // _SUB, _SUB)
    iids2 = item_ids.reshape(B // _SUB, _SUB)

    mesh = plsc.VectorSubcoreMesh(core_axis_name="c", subcore_axis_name="s")

    @functools.partial(
        pl.kernel,
        mesh=mesh,
        out_type=jax.ShapeDtypeStruct((B, 2 * D), jnp.float32),
        scratch_types=[
            pltpu.VMEM((nchunk, _SUB), jnp.int32),
            pltpu.VMEM((ch, D), jnp.float32),
            pltpu.SemaphoreType.DMA,
        ],
    )
    def gk(uids, iids, ut, it, out, idx_v, rows_v, sem):
        wid = lax.axis_index("s") * _NC + lax.axis_index("c")
        base = wid * ch
        for col, ids, tbl in ((0, uids, ut), (D, iids, it)):
            pltpu.sync_copy(ids.at[pl.ds(wid * nchunk, nchunk)], idx_v)
            handles = []
            for j in range(nchunk):
                handles.append(
                    pltpu.async_copy(
                        tbl.at[idx_v.at[j]],
                        rows_v.at[pl.ds(j * _SUB, _SUB)],
                        sem,
                    )
                )
            for h in handles:
                h.wait()
            pltpu.sync_copy(rows_v, out.at[pl.ds(base, ch), pl.ds(col, D)])

    return gk(uids2, iids2, user_table, item_table)


def _genre_body(gid_ref, gt_ref, aw_ref, ge_ref, *, G, NGP, TB):
    f32 = jnp.float32
    bf16 = jnp.bfloat16
    L = G * NGP
    gt = gt_ref[...]
    aw = aw_ref[...]
    lt_row = lax.dot_general(aw, gt, (((1,), (1,)), ((), ())),
                             preferred_element_type=f32)
    lt_tiled = jnp.concatenate([lt_row] * G, axis=1)
    rg = lax.broadcasted_iota(jnp.int32, (G, L), 0)
    rl = lax.broadcasted_iota(jnp.int32, (G, L), 1)
    R = (rg == rl // NGP).astype(bf16)
    gid_rep = lax.dot_general(gid_ref[...].astype(bf16), R,
                              (((1,), (0,)), ((), ())),
                              preferred_element_type=f32)
    lane_t = (lax.broadcasted_iota(jnp.int32, (TB, L), 1) % NGP).astype(f32)
    cmp = gid_rep == lane_t
    mlt = jnp.where(cmp, lt_tiled, 0.0)
    l5 = lax.dot_general(mlt.astype(bf16), R, (((1,), (1,)), ((), ())),
                         preferred_element_type=f32)
    m = jnp.max(l5, axis=1, keepdims=True)
    e5 = jnp.exp(l5 - m)
    w5 = e5 / jnp.sum(e5, axis=1, keepdims=True)
    w_rep = lax.dot_general(w5.astype(bf16), R, (((1,), (0,)), ((), ())),
                            preferred_element_type=f32)
    woh = jnp.where(cmp, w_rep, 0.0)
    gt5 = jnp.concatenate([gt] * G, axis=0).astype(bf16)
    ge_ref[...] = lax.dot_general(woh.astype(bf16), gt5,
                                  (((1,), (0,)), ((), ())),
                                  preferred_element_type=f32).astype(bf16)


def _mlp_body(uei_ref, gemb_ref,
              w0ui_ref, w0g_ref, b0_ref, g0_ref, be0_ref,
              w1_ref, b1_ref, g1_ref, be1_ref,
              w2_ref, b2_ref, g2_ref, be2_ref,
              wp_ref, bp_ref,
              o_ref,
              z0_sc, s0_sc, ss0_sc,
              *, B, TB, NT):
    s = pl.program_id(0)
    bf16 = jnp.bfloat16

    def mm(a, w_ref):
        return lax.dot_general(a.astype(bf16), w_ref[...].astype(bf16),
                               (((1,), (1,)), ((), ())),
                               preferred_element_type=jnp.float32)

    @pl.when(s < NT)
    def _pass0():
        row = pl.ds(pl.multiple_of(s * TB, TB), TB)
        z = mm(uei_ref[...], w0ui_ref)
        z = z + mm(gemb_ref[...], w0g_ref) + b0_ref[...]
        z0_sc[row, :] = z

        @pl.when(s == 0)
        def _():
            s0_sc[...] = jnp.zeros_like(s0_sc)
            ss0_sc[...] = jnp.zeros_like(ss0_sc)

        s0_sc[...] += jnp.sum(z, axis=0, keepdims=True)
        ss0_sc[...] += jnp.sum(z * z, axis=0, keepdims=True)

    def bn_relu(z, s_row, ss_row, g_ref, be_ref):
        mean = s_row / B
        var = ss_row / B - mean * mean
        a = lax.rsqrt(var + _EPS) * g_ref[...]
        c = be_ref[...] - mean * a
        return jnp.maximum(z * a + c, 0.0)

    @pl.when(s == NT)
    def _rest():
        h0 = bn_relu(z0_sc[...], s0_sc[...], ss0_sc[...], g0_ref, be0_ref)
        z1 = mm(h0, w1_ref) + b1_ref[...]
        h1 = bn_relu(z1, jnp.sum(z1, axis=0, keepdims=True),
                     jnp.sum(z1 * z1, axis=0, keepdims=True),
                     g1_ref, be1_ref)
        z2 = mm(h1, w2_ref) + b2_ref[...]
        h2 = bn_relu(z2, jnp.sum(z2, axis=0, keepdims=True),
                     jnp.sum(z2 * z2, axis=0, keepdims=True),
                     g2_ref, be2_ref)
        ot = lax.dot_general(wp_ref[...].astype(bf16), h2.astype(bf16),
                             (((1,), (1,)), ((), ())),
                             preferred_element_type=jnp.float32) + bp_ref[0, 0]
        o_ref[...] = (jax.nn.sigmoid(ot) * 5.0).reshape(1, 1, B)


def kernel(user_ids, item_ids, genre_ids, user_table, item_table, genre_table,
           attn_w, attn_b, W0, b0, gamma0, beta0, W1, b1, gamma1, beta1,
           W2, b2, gamma2, beta2, Wp, bp):
    B = user_ids.shape[0]
    G = genre_ids.shape[1]
    NG, DG = genre_table.shape
    D = user_table.shape[1]
    TB = 8192
    nt = B // TB
    f32 = jnp.float32

    uei = _sc_gather(user_ids.astype(jnp.int32), item_ids.astype(jnp.int32),
                     user_table, item_table)

    NGP = 24
    gt_pad = jnp.zeros((NGP, DG), f32).at[:NG].set(genre_table)
    gid = genre_ids.astype(jnp.int32)

    gemb = pl.pallas_call(
        functools.partial(_genre_body, G=G, NGP=NGP, TB=TB),
        grid=(nt,),
        in_specs=[
            pl.BlockSpec((TB, G), lambda i: (i, 0)),
            pl.BlockSpec((NGP, DG), lambda i: (0, 0)),
            pl.BlockSpec((1, DG), lambda i: (0, 0)),
        ],
        out_specs=pl.BlockSpec((TB, DG), lambda i: (i, 0)),
        out_shape=jax.ShapeDtypeStruct((B, DG), jnp.bfloat16),
    )(gid, gt_pad, attn_w)

    H0, H1, H2 = W0.shape[0], W1.shape[0], W2.shape[0]
    row = lambda v: v.reshape(1, -1)

    p0 = lambda s: (jnp.minimum(s, nt - 1), 0)
    fix = lambda s: (0, 0)
    full = lambda sh: pl.BlockSpec(sh, fix)

    out = pl.pallas_call(
        functools.partial(_mlp_body, B=B, TB=TB, NT=nt),
        grid=(nt + 1,),
        in_specs=[
            pl.BlockSpec((TB, 2 * D), p0),
            pl.BlockSpec((TB, DG), p0),
            full((H0, 2 * D)), full((H0, DG)),
            full((1, H0)), full((1, H0)), full((1, H0)),
            full((H1, H0)), full((1, H1)), full((1, H1)), full((1, H1)),
            full((H2, H1)), full((1, H2)), full((1, H2)), full((1, H2)),
            full((1, H2)), full((1, 1)),
        ],
        out_specs=pl.BlockSpec((1, 1, B), lambda s: (0, 0, 0)),
        out_shape=jax.ShapeDtypeStruct((1, 1, B), f32),
        scratch_shapes=[
            pltpu.VMEM((B, H0), f32),
            pltpu.VMEM((1, H0), f32), pltpu.VMEM((1, H0), f32),
        ],
    )(uei, gemb,
      W0[:, :2 * D], W0[:, 2 * D:],
      row(b0), row(gamma0), row(beta0),
      W1, row(b1), row(gamma1), row(beta1),
      W2, row(b2), row(gamma2), row(beta2),
      Wp, row(bp))

    return out.reshape(B)
